# baseline (speedup 1.0000x reference)
import functools

import jax
import jax.numpy as jnp
from jax import lax
from jax.experimental import pallas as pl
from jax.experimental.pallas import tpu as pltpu
from jax.experimental.pallas import tpu_sc as plsc

_NUM_CORES = 2
_NUM_SUBCORES = 16
_NUM_WORKERS = _NUM_CORES * _NUM_SUBCORES
_CHUNK = 128


_BRO = 4096


def _repack_table(tableT, V, D):
    grid = (V + 2 * _BRO - 1) // (2 * _BRO)

    def repack_body(x_ref, o_ref):
        eye = jnp.eye(D, dtype=jnp.bfloat16)
        o_ref[:, :D] = lax.dot_general(
            x_ref[:, :_BRO].astype(jnp.bfloat16), eye,
            (((0,), (0,)), ((), ())),
            preferred_element_type=jnp.float32,
        )
        o_ref[:, D:] = lax.dot_general(
            x_ref[:, _BRO:].astype(jnp.bfloat16), eye,
            (((0,), (0,)), ((), ())),
            preferred_element_type=jnp.float32,
        )

    return pl.pallas_call(
        repack_body,
        grid=(grid,),
        in_specs=[
            pl.BlockSpec((D, 2 * _BRO), lambda i: (0, i)),
        ],
        out_specs=pl.BlockSpec((_BRO, 2 * D), lambda i: (i, 0)),
        out_shape=jax.ShapeDtypeStruct((grid * _BRO, 2 * D), jnp.float32),
    )(tableT)


def _sc_gather(idx, packed, B, W):
    n_chunks = (B // _NUM_WORKERS) // _CHUNK
    b_per_w = n_chunks * _CHUNK
    mesh = plsc.VectorSubcoreMesh(core_axis_name="c", subcore_axis_name="s")

    @functools.partial(
        pl.kernel,
        out_type=jax.ShapeDtypeStruct((B, W), jnp.float32),
        mesh=mesh,
        scratch_types=[
            pltpu.VMEM((n_chunks, _CHUNK), jnp.int32),
            pltpu.VMEM((b_per_w, W), jnp.float32),
            pltpu.SemaphoreType.DMA,
        ],
    )
    def gather_k(idx_hbm, table_hbm, out_hbm, idx_v, rows_v, sem):
        wid = lax.axis_index("s") * _NUM_CORES + lax.axis_index("c")
        base = wid * b_per_w
        pltpu.sync_copy(idx_hbm.at[wid], idx_v)
        copies = [
            pltpu.async_copy(
                table_hbm.at[idx_v.at[j]],
                rows_v.at[pl.ds(j * _CHUNK, _CHUNK)],
                sem,
            )
            for j in range(n_chunks)
        ]
        for c in copies:
            c.wait()
        pltpu.sync_copy(rows_v, out_hbm.at[pl.ds(base, b_per_w)])

    return gather_k(idx, packed)


def _mlp(rows, hi, W1, b1, W2, b2, B, D, H):
    BM = 2048

    def mlp_body(r_ref, p_ref, w1_ref, b1_ref, w2_ref, b2_ref, o_ref):
        r = r_ref[...]
        is_hi = p_ref[...] != 0
        h = jnp.where(is_hi, r[:, D:], r[:, :D])
        z = jnp.dot(h, w1_ref[...], preferred_element_type=jnp.float32)
        z = z + b1_ref[...]
        z = z * jax.nn.sigmoid(z)
        o_ref[...] = (
            jnp.dot(z, w2_ref[...], preferred_element_type=jnp.float32)
            + b2_ref[...]
        )

    return pl.pallas_call(
        mlp_body,
        grid=(B // BM,),
        in_specs=[
            pl.BlockSpec((BM, 2 * D), lambda i: (i, 0)),
            pl.BlockSpec((BM, 1), lambda i: (i, 0)),
            pl.BlockSpec((D, H), lambda i: (0, 0)),
            pl.BlockSpec((1, H), lambda i: (0, 0)),
            pl.BlockSpec((H, D), lambda i: (0, 0)),
            pl.BlockSpec((1, D), lambda i: (0, 0)),
        ],
        out_specs=pl.BlockSpec((BM, D), lambda i: (i, 0)),
        out_shape=jax.ShapeDtypeStruct((B, D), jnp.float32),
    )(rows, hi, W1, b1, W2, b2)


def kernel(x, table, W1, b1, W2, b2):
    B, = x.shape
    V, D = table.shape
    H = W1.shape[1]
    log_bro = _BRO.bit_length() - 1
    x32 = x.astype(jnp.int32)
    is_hi = (x32 >> log_bro) & 1
    pair_idx = ((x32 >> (log_bro + 1)) * _BRO + (x32 & (_BRO - 1))).reshape(
        _NUM_WORKERS, (B // _NUM_WORKERS) // _CHUNK, _CHUNK
    )
    packed = _repack_table(table.T, V, D)
    rows = _sc_gather(pair_idx, packed, B, 2 * D)
    return _mlp(
        rows, is_hi.reshape(B, 1), W1, b1.reshape(1, H), W2,
        b2.reshape(1, D), B, D, H,
    )


# speedup vs baseline: 1.2510x; 1.2510x over previous
import functools

import jax
import jax.numpy as jnp
from jax import lax
from jax.experimental import pallas as pl
from jax.experimental.pallas import tpu as pltpu
from jax.experimental.pallas import tpu_sc as plsc

_NUM_CORES = 2
_NUM_SUBCORES = 16
_NUM_WORKERS = _NUM_CORES * _NUM_SUBCORES
_CHUNK = 128
_BN = 4096


def _repack_table(tableT, V, D):
    grid = (V + 4 * _BN - 1) // (4 * _BN)

    def repack_body(a_ref, b_ref, c_ref, d_ref, o_ref):
        xcat = jnp.concatenate(
            [a_ref[...], b_ref[...], c_ref[...], d_ref[...]], axis=0
        ).astype(jnp.bfloat16)
        eye = jnp.eye(4 * D, dtype=jnp.bfloat16)
        o_ref[...] = lax.dot_general(
            xcat, eye, (((0,), (0,)), ((), ())),
            preferred_element_type=jnp.float32,
        )

    last_panel = (V - 1) // _BN
    specs = [
        pl.BlockSpec(
            (D, _BN),
            (lambda k: (lambda i: (0, jnp.minimum(4 * i + k, last_panel))))(k),
        )
        for k in range(4)
    ]
    return pl.pallas_call(
        repack_body,
        grid=(grid,),
        in_specs=specs,
        out_specs=pl.BlockSpec((_BN, 4 * D), lambda i: (i, 0)),
        out_shape=jax.ShapeDtypeStruct((grid * _BN, 4 * D), jnp.float32),
    )(tableT, tableT, tableT, tableT)


def _sc_gather(idx, packed, B, W):
    n_chunks = (B // _NUM_WORKERS) // _CHUNK
    b_per_w = n_chunks * _CHUNK
    mesh = plsc.VectorSubcoreMesh(core_axis_name="c", subcore_axis_name="s")

    @functools.partial(
        pl.kernel,
        out_type=jax.ShapeDtypeStruct((B, W), jnp.float32),
        mesh=mesh,
        scratch_types=[
            pltpu.VMEM((n_chunks, _CHUNK), jnp.int32),
            pltpu.VMEM((_CHUNK, W), jnp.float32),
            pltpu.VMEM((_CHUNK, W), jnp.float32),
            pltpu.SemaphoreType.DMA,
            pltpu.SemaphoreType.DMA,
        ],
    )
    def gather_k(idx_hbm, table_hbm, out_hbm, idx_v, buf0, buf1, sem0, sem1):
        wid = lax.axis_index("s") * _NUM_CORES + lax.axis_index("c")
        base = wid * b_per_w
        pltpu.sync_copy(idx_hbm.at[wid], idx_v)
        bufs = (buf0, buf1)
        sems = (sem0, sem1)
        cp = pltpu.async_copy(table_hbm.at[idx_v.at[0]], buf0, sem0)
        for j in range(n_chunks):
            nxt = None
            if j + 1 < n_chunks:
                nxt = pltpu.async_copy(
                    table_hbm.at[idx_v.at[j + 1]],
                    bufs[(j + 1) % 2],
                    sems[(j + 1) % 2],
                )
            cp.wait()
            pltpu.sync_copy(
                bufs[j % 2], out_hbm.at[pl.ds(base + j * _CHUNK, _CHUNK)]
            )
            cp = nxt

    return gather_k(idx, packed)


def _mlp(rows, sub, W1, b1, W2, b2, B, D, H):
    BM = 2048

    def mlp_body(r_ref, s_ref, w1_ref, b1_ref, w2_ref, b2_ref, o_ref):
        r = r_ref[...]
        s = s_ref[...]
        h01 = jnp.where(s == 0, r[:, :D], r[:, D:2 * D])
        h23 = jnp.where(s == 2, r[:, 2 * D:3 * D], r[:, 3 * D:])
        h = jnp.where(s < 2, h01, h23)
        z = jnp.dot(h, w1_ref[...], preferred_element_type=jnp.float32)
        z = z + b1_ref[...]
        z = z * jax.nn.sigmoid(z)
        o_ref[...] = (
            jnp.dot(z, w2_ref[...], preferred_element_type=jnp.float32)
            + b2_ref[...]
        )

    return pl.pallas_call(
        mlp_body,
        grid=(B // BM,),
        in_specs=[
            pl.BlockSpec((BM, 4 * D), lambda i: (i, 0)),
            pl.BlockSpec((BM, 1), lambda i: (i, 0)),
            pl.BlockSpec((D, H), lambda i: (0, 0)),
            pl.BlockSpec((1, H), lambda i: (0, 0)),
            pl.BlockSpec((H, D), lambda i: (0, 0)),
            pl.BlockSpec((1, D), lambda i: (0, 0)),
        ],
        out_specs=pl.BlockSpec((BM, D), lambda i: (i, 0)),
        out_shape=jax.ShapeDtypeStruct((B, D), jnp.float32),
    )(rows, sub, W1, b1, W2, b2)


def kernel(x, table, W1, b1, W2, b2):
    B, = x.shape
    V, D = table.shape
    H = W1.shape[1]
    x32 = x.astype(jnp.int32)
    sub = (x32 >> 12) & 3
    row_idx = ((x32 >> 14) * _BN + (x32 & (_BN - 1))).reshape(
        _NUM_WORKERS, (B // _NUM_WORKERS) // _CHUNK, _CHUNK
    )
    packed = _repack_table(table.T, V, D)
    rows = _sc_gather(row_idx, packed, B, 4 * D)
    return _mlp(
        rows, sub.reshape(B, 1), W1, b1.reshape(1, H), W2,
        b2.reshape(1, D), B, D, H,
    )


# speedup vs baseline: 1.2556x; 1.0037x over previous
import functools

import jax
import jax.numpy as jnp
from jax import lax
from jax.experimental import pallas as pl
from jax.experimental.pallas import tpu as pltpu
from jax.experimental.pallas import tpu_sc as plsc

_NUM_CORES = 2
_NUM_SUBCORES = 16
_NUM_WORKERS = _NUM_CORES * _NUM_SUBCORES
_CHUNK = 128
_BN = 4096


def _repack_table(tableT, V, D):
    grid = (V + 4 * _BN - 1) // (4 * _BN)

    def repack_body(a_ref, b_ref, c_ref, d_ref, o_ref):
        xcat = jnp.concatenate(
            [a_ref[...], b_ref[...], c_ref[...], d_ref[...]], axis=0
        ).astype(jnp.bfloat16)
        eye = jnp.eye(4 * D, dtype=jnp.bfloat16)
        t = lax.dot_general(
            xcat, eye, (((0,), (0,)), ((), ())),
            preferred_element_type=jnp.float32,
        )
        r32 = lax.bitcast_convert_type(t, jnp.uint32)
        rnd = (r32 + 0x7FFF + ((r32 >> 16) & 1)) >> 16
        ab = rnd[:, :D] | (rnd[:, D:2 * D] << 16)
        cd = rnd[:, 2 * D:3 * D] | (rnd[:, 3 * D:] << 16)
        o_ref[...] = lax.bitcast_convert_type(
            jnp.concatenate([ab, cd], axis=1), jnp.float32
        )

    last_panel = (V - 1) // _BN
    specs = [
        pl.BlockSpec(
            (D, _BN),
            (lambda k: (lambda i: (0, jnp.minimum(4 * i + k, last_panel))))(k),
        )
        for k in range(4)
    ]
    return pl.pallas_call(
        repack_body,
        grid=(grid,),
        in_specs=specs,
        out_specs=pl.BlockSpec((_BN, 2 * D), lambda i: (i, 0)),
        out_shape=jax.ShapeDtypeStruct((grid * _BN, 2 * D), jnp.float32),
    )(tableT, tableT, tableT, tableT)


def _sc_gather(idx, packed, B, W):
    n_chunks = (B // _NUM_WORKERS) // _CHUNK
    b_per_w = n_chunks * _CHUNK
    mesh = plsc.VectorSubcoreMesh(core_axis_name="c", subcore_axis_name="s")

    @functools.partial(
        pl.kernel,
        out_type=jax.ShapeDtypeStruct((B, W), jnp.float32),
        mesh=mesh,
        scratch_types=[
            pltpu.VMEM((n_chunks, _CHUNK), jnp.int32),
            pltpu.VMEM((_CHUNK, W), jnp.float32),
            pltpu.VMEM((_CHUNK, W), jnp.float32),
            pltpu.SemaphoreType.DMA,
            pltpu.SemaphoreType.DMA,
        ],
    )
    def gather_k(idx_hbm, table_hbm, out_hbm, idx_v, buf0, buf1, sem0, sem1):
        wid = lax.axis_index("s") * _NUM_CORES + lax.axis_index("c")
        base = wid * b_per_w
        pltpu.sync_copy(idx_hbm.at[wid], idx_v)
        bufs = (buf0, buf1)
        sems = (sem0, sem1)
        cp = pltpu.async_copy(table_hbm.at[idx_v.at[0]], buf0, sem0)
        for j in range(n_chunks):
            nxt = None
            if j + 1 < n_chunks:
                nxt = pltpu.async_copy(
                    table_hbm.at[idx_v.at[j + 1]],
                    bufs[(j + 1) % 2],
                    sems[(j + 1) % 2],
                )
            cp.wait()
            pltpu.sync_copy(
                bufs[j % 2], out_hbm.at[pl.ds(base + j * _CHUNK, _CHUNK)]
            )
            cp = nxt

    return gather_k(idx, packed)


def _mlp(rows, sub, W1, b1, W2, b2, B, D, H):
    BM = 2048

    def mlp_body(r_ref, s_ref, w1_ref, b1_ref, w2_ref, b2_ref, o_ref):
        w = lax.bitcast_convert_type(r_ref[...], jnp.uint32)
        s = s_ref[...]
        word = jnp.where(s < 2, w[:, :D], w[:, D:])
        u = jnp.where((s & 1) == 1, word >> 16, word & 0xFFFF)
        h = lax.bitcast_convert_type(u << 16, jnp.float32)
        z = jnp.dot(h, w1_ref[...], preferred_element_type=jnp.float32)
        z = z + b1_ref[...]
        z = z * jax.nn.sigmoid(z)
        o_ref[...] = (
            jnp.dot(z, w2_ref[...], preferred_element_type=jnp.float32)
            + b2_ref[...]
        )

    return pl.pallas_call(
        mlp_body,
        grid=(B // BM,),
        in_specs=[
            pl.BlockSpec((BM, 2 * D), lambda i: (i, 0)),
            pl.BlockSpec((BM, 1), lambda i: (i, 0)),
            pl.BlockSpec((D, H), lambda i: (0, 0)),
            pl.BlockSpec((1, H), lambda i: (0, 0)),
            pl.BlockSpec((H, D), lambda i: (0, 0)),
            pl.BlockSpec((1, D), lambda i: (0, 0)),
        ],
        out_specs=pl.BlockSpec((BM, D), lambda i: (i, 0)),
        out_shape=jax.ShapeDtypeStruct((B, D), jnp.float32),
    )(rows, sub, W1, b1, W2, b2)


def kernel(x, table, W1, b1, W2, b2):
    B, = x.shape
    V, D = table.shape
    H = W1.shape[1]
    x32 = x.astype(jnp.int32)
    sub = (x32 >> 12) & 3
    row_idx = ((x32 >> 14) * _BN + (x32 & (_BN - 1))).reshape(
        _NUM_WORKERS, (B // _NUM_WORKERS) // _CHUNK, _CHUNK
    )
    packed = _repack_table(table.T, V, D)
    rows = _sc_gather(row_idx, packed, B, 2 * D)
    return _mlp(
        rows, sub.reshape(B, 1), W1, b1.reshape(1, H), W2,
        b2.reshape(1, D), B, D, H,
    )


# speedup vs baseline: 1.3860x; 1.1039x over previous
import functools

import jax
import jax.numpy as jnp
from jax import lax
from jax.experimental import pallas as pl
from jax.experimental.pallas import tpu as pltpu
from jax.experimental.pallas import tpu_sc as plsc

_NUM_CORES = 2
_NUM_SUBCORES = 16
_NUM_WORKERS = _NUM_CORES * _NUM_SUBCORES
_CHUNK = 128
_BN = 8192


def _repack_table(tableT, V, D):
    grid = (V + 4 * _BN - 1) // (4 * _BN)

    def repack_body(a_ref, b_ref, c_ref, d_ref, o_ref):
        xcat = jnp.concatenate(
            [a_ref[...], b_ref[...], c_ref[...], d_ref[...]], axis=0
        ).astype(jnp.bfloat16)
        eye = jnp.eye(4 * D, dtype=jnp.bfloat16)
        t = lax.dot_general(
            xcat, eye, (((0,), (0,)), ((), ())),
            preferred_element_type=jnp.float32,
        )
        r32 = lax.bitcast_convert_type(t, jnp.uint32)
        rnd = (r32 + 0x7FFF + ((r32 >> 16) & 1)) >> 16
        ab = rnd[:, :D] | (rnd[:, D:2 * D] << 16)
        cd = rnd[:, 2 * D:3 * D] | (rnd[:, 3 * D:] << 16)
        o_ref[...] = lax.bitcast_convert_type(
            jnp.concatenate([ab, cd], axis=1), jnp.float32
        )

    last_panel = (V - 1) // _BN
    specs = [
        pl.BlockSpec(
            (D, _BN),
            (lambda k: (lambda i: (0, jnp.minimum(4 * i + k, last_panel))))(k),
        )
        for k in range(4)
    ]
    return pl.pallas_call(
        repack_body,
        grid=(grid,),
        in_specs=specs,
        out_specs=pl.BlockSpec((_BN, 2 * D), lambda i: (i, 0)),
        out_shape=jax.ShapeDtypeStruct((grid * _BN, 2 * D), jnp.float32),
    )(tableT, tableT, tableT, tableT)


def _sc_gather(idx, packed, B, W):
    n_chunks = (B // _NUM_WORKERS) // _CHUNK
    b_per_w = n_chunks * _CHUNK
    mesh = plsc.VectorSubcoreMesh(core_axis_name="c", subcore_axis_name="s")

    @functools.partial(
        pl.kernel,
        out_type=jax.ShapeDtypeStruct((B, W), jnp.float32),
        mesh=mesh,
        scratch_types=[
            pltpu.VMEM((n_chunks, _CHUNK), jnp.int32),
            pltpu.VMEM((_CHUNK, W), jnp.float32),
            pltpu.VMEM((_CHUNK, W), jnp.float32),
            pltpu.SemaphoreType.DMA,
            pltpu.SemaphoreType.DMA,
        ],
    )
    def gather_k(idx_hbm, table_hbm, out_hbm, idx_v, buf0, buf1, sem0, sem1):
        wid = lax.axis_index("s") * _NUM_CORES + lax.axis_index("c")
        base = wid * b_per_w
        pltpu.sync_copy(idx_hbm.at[wid], idx_v)
        bufs = (buf0, buf1)
        sems = (sem0, sem1)
        cp = pltpu.async_copy(table_hbm.at[idx_v.at[0]], buf0, sem0)
        for j in range(n_chunks):
            nxt = None
            if j + 1 < n_chunks:
                nxt = pltpu.async_copy(
                    table_hbm.at[idx_v.at[j + 1]],
                    bufs[(j + 1) % 2],
                    sems[(j + 1) % 2],
                )
            cp.wait()
            pltpu.sync_copy(
                bufs[j % 2], out_hbm.at[pl.ds(base + j * _CHUNK, _CHUNK)]
            )
            cp = nxt

    return gather_k(idx, packed)


def _mlp(rows, sub, W1, b1, W2, b2, B, D, H):
    BM = 2048

    def mlp_body(r_ref, s_ref, w1_ref, b1_ref, w2_ref, b2_ref, o_ref):
        w = lax.bitcast_convert_type(r_ref[...], jnp.uint32)
        s = s_ref[...]
        word = jnp.where(s < 2, w[:, :D], w[:, D:])
        u = jnp.where((s & 1) == 1, word >> 16, word & 0xFFFF)
        h = lax.bitcast_convert_type(u << 16, jnp.float32)
        z = jnp.dot(h, w1_ref[...], preferred_element_type=jnp.float32)
        z = z + b1_ref[...]
        z = z * jax.nn.sigmoid(z)
        o_ref[...] = (
            jnp.dot(z, w2_ref[...], preferred_element_type=jnp.float32)
            + b2_ref[...]
        )

    return pl.pallas_call(
        mlp_body,
        grid=(B // BM,),
        in_specs=[
            pl.BlockSpec((BM, 2 * D), lambda i: (i, 0)),
            pl.BlockSpec((BM, 1), lambda i: (i, 0)),
            pl.BlockSpec((D, H), lambda i: (0, 0)),
            pl.BlockSpec((1, H), lambda i: (0, 0)),
            pl.BlockSpec((H, D), lambda i: (0, 0)),
            pl.BlockSpec((1, D), lambda i: (0, 0)),
        ],
        out_specs=pl.BlockSpec((BM, D), lambda i: (i, 0)),
        out_shape=jax.ShapeDtypeStruct((B, D), jnp.float32),
    )(rows, sub, W1, b1, W2, b2)


def kernel(x, table, W1, b1, W2, b2):
    B, = x.shape
    V, D = table.shape
    H = W1.shape[1]
    log_bn = _BN.bit_length() - 1
    x32 = x.astype(jnp.int32)
    sub = (x32 >> log_bn) & 3
    row_idx = ((x32 >> (log_bn + 2)) * _BN + (x32 & (_BN - 1))).reshape(
        _NUM_WORKERS, (B // _NUM_WORKERS) // _CHUNK, _CHUNK
    )
    packed = _repack_table(table.T, V, D)
    rows = _sc_gather(row_idx, packed, B, 2 * D)
    return _mlp(
        rows, sub.reshape(B, 1), W1, b1.reshape(1, H), W2,
        b2.reshape(1, D), B, D, H,
    )


# speedup vs baseline: 1.4238x; 1.0272x over previous
import functools

import jax
import jax.numpy as jnp
from jax import lax
from jax.experimental import pallas as pl
from jax.experimental.pallas import tpu as pltpu
from jax.experimental.pallas import tpu_sc as plsc

_NUM_CORES = 2
_NUM_SUBCORES = 16
_NUM_WORKERS = _NUM_CORES * _NUM_SUBCORES
_CHUNK = 128
_BN = 8192


def _repack_table(tableT, V, D):
    grid = (V + 4 * _BN - 1) // (4 * _BN)

    def repack_body(a_ref, b_ref, c_ref, d_ref, o_ref):
        xcat = jnp.concatenate(
            [a_ref[...], b_ref[...], c_ref[...], d_ref[...]], axis=0
        ).astype(jnp.bfloat16)
        eye = jnp.eye(4 * D, dtype=jnp.bfloat16)
        t = lax.dot_general(
            xcat, eye, (((0,), (0,)), ((), ())),
            preferred_element_type=jnp.float32,
        )
        r32 = lax.bitcast_convert_type(t, jnp.uint32)
        rnd = (r32 + 0x7FFF + ((r32 >> 16) & 1)) >> 16
        ab = rnd[:, :D] | (rnd[:, D:2 * D] << 16)
        cd = rnd[:, 2 * D:3 * D] | (rnd[:, 3 * D:] << 16)
        o_ref[...] = lax.bitcast_convert_type(
            jnp.concatenate([ab, cd], axis=1), jnp.float32
        )

    last_panel = (V - 1) // _BN
    specs = [
        pl.BlockSpec(
            (D, _BN),
            (lambda k: (lambda i: (0, jnp.minimum(4 * i + k, last_panel))))(k),
        )
        for k in range(4)
    ]
    return pl.pallas_call(
        repack_body,
        grid=(grid,),
        in_specs=specs,
        out_specs=pl.BlockSpec((_BN, 2 * D), lambda i: (i, 0)),
        out_shape=jax.ShapeDtypeStruct((grid * _BN, 2 * D), jnp.float32),
    )(tableT, tableT, tableT, tableT)


def _sc_gather(idx, packed, B, W):
    n_chunks = (B // _NUM_WORKERS) // _CHUNK
    b_per_w = n_chunks * _CHUNK
    mesh = plsc.VectorSubcoreMesh(core_axis_name="c", subcore_axis_name="s")

    @functools.partial(
        pl.kernel,
        out_type=jax.ShapeDtypeStruct((B, W), jnp.float32),
        mesh=mesh,
        scratch_types=[
            pltpu.VMEM((n_chunks, _CHUNK), jnp.int32),
            pltpu.VMEM((_CHUNK, W), jnp.float32),
            pltpu.VMEM((_CHUNK, W), jnp.float32),
            pltpu.SemaphoreType.DMA,
            pltpu.SemaphoreType.DMA,
        ],
    )
    def gather_k(idx_hbm, table_hbm, out_hbm, idx_v, buf0, buf1, sem0, sem1):
        wid = lax.axis_index("s") * _NUM_CORES + lax.axis_index("c")
        base = wid * b_per_w
        pltpu.sync_copy(idx_hbm.at[wid], idx_v)
        bufs = (buf0, buf1)
        sems = (sem0, sem1)
        cp = pltpu.async_copy(table_hbm.at[idx_v.at[0]], buf0, sem0)
        for j in range(n_chunks):
            nxt = None
            if j + 1 < n_chunks:
                nxt = pltpu.async_copy(
                    table_hbm.at[idx_v.at[j + 1]],
                    bufs[(j + 1) % 2],
                    sems[(j + 1) % 2],
                )
            cp.wait()
            pltpu.sync_copy(
                bufs[j % 2], out_hbm.at[pl.ds(base + j * _CHUNK, _CHUNK)]
            )
            cp = nxt

    return gather_k(idx, packed)


def _mlp(rows, sub, W1, b1, W2, b2, B, D, H):
    BM = 2048

    def mlp_body(r_ref, s_ref, w1_ref, b1_ref, w2_ref, b2_ref, o_ref):
        w = lax.bitcast_convert_type(r_ref[...], jnp.uint32)
        s = s_ref[...]
        word = jnp.where(s < 2, w[:, :D], w[:, D:])
        u = jnp.where((s & 1) == 1, word >> 16, word & 0xFFFF)
        h = lax.bitcast_convert_type(u << 16, jnp.float32)
        zT = lax.dot_general(
            w1_ref[...], h, (((0,), (1,)), ((), ())),
            preferred_element_type=jnp.float32,
        )
        zT = zT + b1_ref[...]
        zT = zT * jax.nn.sigmoid(zT)
        o_ref[...] = (
            lax.dot_general(
                w2_ref[...], zT, (((0,), (0,)), ((), ())),
                preferred_element_type=jnp.float32,
            )
            + b2_ref[...]
        )

    return pl.pallas_call(
        mlp_body,
        grid=(B // BM,),
        in_specs=[
            pl.BlockSpec((BM, 2 * D), lambda i: (i, 0)),
            pl.BlockSpec((BM, 1), lambda i: (i, 0)),
            pl.BlockSpec((D, H), lambda i: (0, 0)),
            pl.BlockSpec((H, 1), lambda i: (0, 0)),
            pl.BlockSpec((H, D), lambda i: (0, 0)),
            pl.BlockSpec((D, 1), lambda i: (0, 0)),
        ],
        out_specs=pl.BlockSpec((D, BM), lambda i: (0, i)),
        out_shape=jax.ShapeDtypeStruct((D, B), jnp.float32),
    )(rows, sub, W1, b1, W2, b2)


def kernel(x, table, W1, b1, W2, b2):
    B, = x.shape
    V, D = table.shape
    H = W1.shape[1]
    log_bn = _BN.bit_length() - 1
    x32 = x.astype(jnp.int32)
    sub = (x32 >> log_bn) & 3
    row_idx = ((x32 >> (log_bn + 2)) * _BN + (x32 & (_BN - 1))).reshape(
        _NUM_WORKERS, (B // _NUM_WORKERS) // _CHUNK, _CHUNK
    )
    packed = _repack_table(table.T, V, D)
    rows = _sc_gather(row_idx, packed, B, 2 * D)
    outT = _mlp(
        rows, sub.reshape(B, 1), W1, b1.reshape(H, 1), W2,
        b2.reshape(D, 1), B, D, H,
    )
    return outT.T


# speedup vs baseline: 1.4282x; 1.0031x over previous
import functools

import jax
import jax.numpy as jnp
from jax import lax
from jax.experimental import pallas as pl
from jax.experimental.pallas import tpu as pltpu
from jax.experimental.pallas import tpu_sc as plsc

_NUM_CORES = 2
_NUM_SUBCORES = 16
_NUM_WORKERS = _NUM_CORES * _NUM_SUBCORES
_CHUNK = 128
_BN = 8192


def _repack_table(tableT, V, D):
    grid = (V + 4 * _BN - 1) // (4 * _BN)

    def repack_body(a_ref, b_ref, c_ref, d_ref, o_ref):
        xcat = jnp.concatenate(
            [a_ref[...], b_ref[...], c_ref[...], d_ref[...]], axis=0
        ).astype(jnp.bfloat16)
        eye = jnp.eye(4 * D, dtype=jnp.bfloat16)
        t = lax.dot_general(
            xcat, eye, (((0,), (0,)), ((), ())),
            preferred_element_type=jnp.float32,
        )
        r32 = lax.bitcast_convert_type(t, jnp.uint32)
        rnd = (r32 + 0x7FFF + ((r32 >> 16) & 1)) >> 16
        ab = rnd[:, :D] | (rnd[:, D:2 * D] << 16)
        cd = rnd[:, 2 * D:3 * D] | (rnd[:, 3 * D:] << 16)
        o_ref[...] = lax.bitcast_convert_type(
            jnp.concatenate([ab, cd], axis=1), jnp.float32
        )

    last_panel = (V - 1) // _BN
    specs = [
        pl.BlockSpec(
            (D, _BN),
            (lambda k: (lambda i: (0, jnp.minimum(4 * i + k, last_panel))))(k),
        )
        for k in range(4)
    ]
    return pl.pallas_call(
        repack_body,
        grid=(grid,),
        in_specs=specs,
        out_specs=pl.BlockSpec((_BN, 2 * D), lambda i: (i, 0)),
        out_shape=jax.ShapeDtypeStruct((grid * _BN, 2 * D), jnp.float32),
    )(tableT, tableT, tableT, tableT)


def _sc_gather(idx, packed, B, W):
    n_chunks = (B // _NUM_WORKERS) // _CHUNK
    b_per_w = n_chunks * _CHUNK
    mesh = plsc.VectorSubcoreMesh(core_axis_name="c", subcore_axis_name="s")

    @functools.partial(
        pl.kernel,
        out_type=jax.ShapeDtypeStruct((B, W), jnp.float32),
        mesh=mesh,
        scratch_types=[
            pltpu.VMEM((n_chunks, _CHUNK), jnp.int32),
            pltpu.VMEM((_CHUNK, W), jnp.float32),
            pltpu.VMEM((_CHUNK, W), jnp.float32),
            pltpu.SemaphoreType.DMA,
            pltpu.SemaphoreType.DMA,
        ],
    )
    def gather_k(idx_hbm, table_hbm, out_hbm, idx_v, buf0, buf1, sem0, sem1):
        wid = lax.axis_index("s") * _NUM_CORES + lax.axis_index("c")
        base = wid * b_per_w
        pltpu.sync_copy(idx_hbm.at[wid], idx_v)
        bufs = (buf0, buf1)
        sems = (sem0, sem1)
        cp = pltpu.async_copy(table_hbm.at[idx_v.at[0]], buf0, sem0)
        for j in range(n_chunks):
            nxt = None
            if j + 1 < n_chunks:
                nxt = pltpu.async_copy(
                    table_hbm.at[idx_v.at[j + 1]],
                    bufs[(j + 1) % 2],
                    sems[(j + 1) % 2],
                )
            cp.wait()
            pltpu.sync_copy(
                bufs[j % 2], out_hbm.at[pl.ds(base + j * _CHUNK, _CHUNK)]
            )
            cp = nxt

    return gather_k(idx, packed)


def _mlp(rows, sub, W1, b1, W2, b2, B, D, H):
    BM = 4096

    def mlp_body(r_ref, s_ref, w1_ref, b1_ref, w2_ref, b2_ref, o_ref):
        w = lax.bitcast_convert_type(r_ref[...], jnp.uint32)
        s = s_ref[...]
        word = jnp.where(s < 2, w[:, :D], w[:, D:])
        u = jnp.where((s & 1) == 1, word >> 16, word & 0xFFFF)
        h = lax.bitcast_convert_type(u << 16, jnp.float32)
        zT = lax.dot_general(
            w1_ref[...], h, (((0,), (1,)), ((), ())),
            preferred_element_type=jnp.float32,
        )
        zT = zT + b1_ref[...]
        zT = zT * jax.nn.sigmoid(zT)
        o_ref[...] = (
            lax.dot_general(
                w2_ref[...], zT, (((0,), (0,)), ((), ())),
                preferred_element_type=jnp.float32,
            )
            + b2_ref[...]
        )

    return pl.pallas_call(
        mlp_body,
        grid=(B // BM,),
        in_specs=[
            pl.BlockSpec((BM, 2 * D), lambda i: (i, 0)),
            pl.BlockSpec((BM, 1), lambda i: (i, 0)),
            pl.BlockSpec((D, H), lambda i: (0, 0)),
            pl.BlockSpec((H, 1), lambda i: (0, 0)),
            pl.BlockSpec((H, D), lambda i: (0, 0)),
            pl.BlockSpec((D, 1), lambda i: (0, 0)),
        ],
        out_specs=pl.BlockSpec((D, BM), lambda i: (0, i)),
        out_shape=jax.ShapeDtypeStruct((D, B), jnp.float32),
    )(rows, sub, W1, b1, W2, b2)


def kernel(x, table, W1, b1, W2, b2):
    B, = x.shape
    V, D = table.shape
    H = W1.shape[1]
    log_bn = _BN.bit_length() - 1
    x32 = x.astype(jnp.int32)
    sub = (x32 >> log_bn) & 3
    row_idx = ((x32 >> (log_bn + 2)) * _BN + (x32 & (_BN - 1))).reshape(
        _NUM_WORKERS, (B // _NUM_WORKERS) // _CHUNK, _CHUNK
    )
    packed = _repack_table(table.T, V, D)
    rows = _sc_gather(row_idx, packed, B, 2 * D)
    outT = _mlp(
        rows, sub.reshape(B, 1), W1, b1.reshape(H, 1), W2,
        b2.reshape(D, 1), B, D, H,
    )
    return outT.T
